# baseline (speedup 1.0000x reference)
import functools

import jax
import jax.numpy as jnp
from jax import lax
from jax.experimental import pallas as pl
from jax.experimental.pallas import tpu as pltpu
from jax.experimental.pallas import tpu_sc as plsc

_VOCAB = 100000
_DIM = 768
_B = 4 * 2048
_NC = 2
_NS = 16
_NW = _NC * _NS
_BPW = _B // _NW
_CHUNK = 64
_NCHUNK = _BPW // _CHUNK


@jax.jit
def _sc_gather3(W0, W1, W2, idx_flat):
    mesh = plsc.VectorSubcoreMesh(
        core_axis_name="c", subcore_axis_name="s", num_cores=_NC,
        num_subcores=_NS)
    out_type = [jax.ShapeDtypeStruct((_B, _DIM), jnp.float32)] * 3

    @functools.partial(
        pl.kernel,
        mesh=mesh,
        out_type=out_type,
        scratch_types=[
            pltpu.VMEM((_BPW,), jnp.int32),
            pltpu.VMEM((_CHUNK, _DIM), jnp.float32),
            pltpu.VMEM((_CHUNK, _DIM), jnp.float32),
            pltpu.SemaphoreType.DMA,
            pltpu.SemaphoreType.DMA,
            pltpu.SemaphoreType.DMA,
            pltpu.SemaphoreType.DMA,
        ],
    )
    def body(w0, w1, w2, idx_hbm, o0, o1, o2, idx_v, buf0, buf1,
             gsem0, gsem1, wsem0, wsem1):
        wid = lax.axis_index("s") * _NC + lax.axis_index("c")
        base = wid * _BPW
        pltpu.sync_copy(idx_hbm.at[pl.ds(base, _BPW)], idx_v)

        tables = (w0, w1, w2)
        outs = (o0, o1, o2)
        bufs = (buf0, buf1)
        gsems = (gsem0, gsem1)
        wsems = (wsem0, wsem1)
        tasks = [(t, c) for t in range(3) for c in range(_NCHUNK)]
        n = len(tasks)

        def start_gather(i):
            t, c = tasks[i]
            return pltpu.async_copy(
                tables[t].at[idx_v.at[pl.ds(c * _CHUNK, _CHUNK)]],
                bufs[i % 2], gsems[i % 2])

        pend_w = [None, None]
        cp = start_gather(0)
        for i, (t, c) in enumerate(tasks):
            cp.wait()
            wcp = pltpu.async_copy(
                bufs[i % 2], outs[t].at[pl.ds(base + c * _CHUNK, _CHUNK)],
                wsems[i % 2])
            if i + 1 < n:
                b = (i + 1) % 2
                if pend_w[b] is not None:
                    pend_w[b].wait()
                cp = start_gather(i + 1)
            pend_w[i % 2] = wcp
        for b in range(2):
            if pend_w[b] is not None:
                pend_w[b].wait()

    return body(W0, W1, W2, idx_flat)


def kernel(W0, W1, W2, inputs):
    idx_flat = inputs.reshape(-1).astype(jnp.int32)
    e0, e1, e2 = _sc_gather3(W0, W1, W2, idx_flat)
    shape = inputs.shape + (_DIM,)
    e0 = e0.reshape(shape)
    e1 = e1.reshape(shape)
    e2 = e2.reshape(shape)
    return (e0, e1, e2, e2, e1, e0)


# speedup vs baseline: 1.0241x; 1.0241x over previous
import functools

import jax
import jax.numpy as jnp
from jax import lax
from jax.experimental import pallas as pl
from jax.experimental.pallas import tpu as pltpu
from jax.experimental.pallas import tpu_sc as plsc

_VOCAB = 100000
_DIM = 768
_B = 4 * 2048
_NC = 2
_NS = 16
_NW = _NC * _NS
_BPW = _B // _NW
_CHUNK = 32
_NCHUNK = _BPW // _CHUNK
_NBUF = 4
_AHEAD = 2


@jax.jit
def _sc_gather3(W0, W1, W2, idx_flat):
    mesh = plsc.VectorSubcoreMesh(
        core_axis_name="c", subcore_axis_name="s", num_cores=_NC,
        num_subcores=_NS)
    out_type = [jax.ShapeDtypeStruct((_B, _DIM), jnp.float32)] * 3

    @functools.partial(
        pl.kernel,
        mesh=mesh,
        out_type=out_type,
        scratch_types=(
            [pltpu.VMEM((_BPW,), jnp.int32)]
            + [pltpu.VMEM((_CHUNK, _DIM), jnp.float32)] * _NBUF
            + [pltpu.SemaphoreType.DMA] * (2 * _NBUF)
        ),
    )
    def body(w0, w1, w2, idx_hbm, o0, o1, o2, idx_v, *rest):
        bufs = rest[:_NBUF]
        gsems = rest[_NBUF:2 * _NBUF]
        wsems = rest[2 * _NBUF:]
        wid = lax.axis_index("s") * _NC + lax.axis_index("c")
        base = wid * _BPW
        pltpu.sync_copy(idx_hbm.at[pl.ds(base, _BPW)], idx_v)

        tables = (w0, w1, w2)
        outs = (o0, o1, o2)
        tasks = [(t, c) for t in range(3) for c in range(_NCHUNK)]
        n = len(tasks)

        def start_gather(i):
            t, c = tasks[i]
            b = i % _NBUF
            return pltpu.async_copy(
                tables[t].at[idx_v.at[pl.ds(c * _CHUNK, _CHUNK)]],
                bufs[b], gsems[b])

        pend_g = [None] * _NBUF
        pend_w = [None] * _NBUF
        for j in range(min(_AHEAD, n)):
            pend_g[j % _NBUF] = start_gather(j)
        for i, (t, c) in enumerate(tasks):
            b = i % _NBUF
            pend_g[b].wait()
            pend_w[b] = pltpu.async_copy(
                bufs[b], outs[t].at[pl.ds(base + c * _CHUNK, _CHUNK)],
                wsems[b])
            k = i + _AHEAD
            if k < n:
                bk = k % _NBUF
                if pend_w[bk] is not None:
                    pend_w[bk].wait()
                pend_g[bk] = start_gather(k)
        for b in range(_NBUF):
            if pend_w[b] is not None:
                pend_w[b].wait()

    return body(W0, W1, W2, idx_flat)


def kernel(W0, W1, W2, inputs):
    idx_flat = inputs.reshape(-1).astype(jnp.int32)
    e0, e1, e2 = _sc_gather3(W0, W1, W2, idx_flat)
    shape = inputs.shape + (_DIM,)
    e0 = e0.reshape(shape)
    e1 = e1.reshape(shape)
    e2 = e2.reshape(shape)
    return (e0, e1, e2, e2, e1, e0)
